# Initial kernel scaffold; baseline (speedup 1.0000x reference)
import jax
import jax.numpy as jnp
from jax.experimental import pallas as pl


def kernel(word_indices, embeddings):
    raise NotImplementedError("write your pallas kernel here")


# speedup vs baseline: 1.8453x; 1.8453x over previous
import functools

import jax
import jax.numpy as jnp
from jax import lax
from jax.experimental import pallas as pl
from jax.experimental.pallas import tpu as pltpu
from jax.experimental.pallas import tpu_sc as plsc

_SUB = 128
_CH = 1024


@functools.lru_cache(maxsize=None)
def _build_gather(B, V, D):
    info = plsc.get_sparse_core_info()
    nc, ns = info.num_cores, info.num_subcores
    nw = nc * ns
    bpw = B // nw
    assert B % nw == 0 and bpw % _CH == 0 and _CH % _SUB == 0
    g = _CH // _SUB
    nchunks = bpw // _CH
    mesh = plsc.VectorSubcoreMesh(core_axis_name="c", subcore_axis_name="s")

    @functools.partial(
        pl.kernel,
        mesh=mesh,
        out_type=jax.ShapeDtypeStruct((B, D), jnp.float32),
        scratch_types=[
            pltpu.VMEM((_CH,), jnp.int32),
            pltpu.VMEM((_CH, D), jnp.float32),
            pltpu.SemaphoreType.DMA,
        ],
        compiler_params=pltpu.CompilerParams(use_tc_tiling_on_sc=False),
    )
    def gather_kernel(idx_hbm, table_hbm, out_hbm, idx_v, rows_v, sem):
        wid = lax.axis_index("s") * nc + lax.axis_index("c")
        base = wid * bpw

        def chunk(i, carry):
            off = base + i * _CH
            pltpu.sync_copy(idx_hbm.at[pl.ds(off, _CH)], idx_v)
            copies = [
                pltpu.async_copy(
                    table_hbm.at[idx_v.at[pl.ds(j * _SUB, _SUB)]],
                    rows_v.at[pl.ds(j * _SUB, _SUB)],
                    sem,
                )
                for j in range(g)
            ]
            for cp in copies:
                cp.wait()
            pltpu.sync_copy(rows_v, out_hbm.at[pl.ds(off, _CH)])
            return carry

        lax.fori_loop(0, nchunks, chunk, 0)

    return gather_kernel


def kernel(word_indices, embeddings):
    batch, hist = word_indices.shape
    vocab, d = embeddings.shape
    idx = word_indices.reshape(-1).astype(jnp.int32)
    out = _build_gather(batch * hist, vocab, d)(idx, embeddings)
    return out.reshape(batch, hist, d)


# speedup vs baseline: 1.8770x; 1.0172x over previous
import functools

import jax
import jax.numpy as jnp
from jax import lax
from jax.experimental import pallas as pl
from jax.experimental.pallas import tpu as pltpu
from jax.experimental.pallas import tpu_sc as plsc

_SUB = 128
_CH = 512


@functools.lru_cache(maxsize=None)
def _build_gather(B, V, D):
    info = plsc.get_sparse_core_info()
    nc, ns = info.num_cores, info.num_subcores
    nw = nc * ns
    bpw = B // nw
    g = _CH // _SUB
    nchunks = bpw // _CH
    assert B % nw == 0 and bpw % _CH == 0 and _CH % _SUB == 0
    assert nchunks % 2 == 0
    mesh = plsc.VectorSubcoreMesh(core_axis_name="c", subcore_axis_name="s")

    @functools.partial(
        pl.kernel,
        mesh=mesh,
        out_type=jax.ShapeDtypeStruct((B, D), jnp.float32),
        scratch_types=[
            pltpu.VMEM((bpw,), jnp.int32),
            pltpu.VMEM((2, _CH, D), jnp.float32),
            pltpu.SemaphoreType.DMA,
            pltpu.SemaphoreType.DMA,
        ],
        compiler_params=pltpu.CompilerParams(use_tc_tiling_on_sc=False),
    )
    def gather_kernel(idx_hbm, table_hbm, out_hbm, idx_v, rows_v, sem0, sem1):
        sems = (sem0, sem1)
        wid = lax.axis_index("s") * nc + lax.axis_index("c")
        base = wid * bpw
        pltpu.sync_copy(idx_hbm.at[pl.ds(base, bpw)], idx_v)

        def fire(chunk, slot, sem):
            for j in range(g):
                pltpu.async_copy(
                    table_hbm.at[idx_v.at[pl.ds(chunk * _CH + j * _SUB, _SUB)]],
                    rows_v.at[slot].at[pl.ds(j * _SUB, _SUB)],
                    sem,
                )

        def drain(slot, sem):
            for j in range(g):
                pltpu.make_async_copy(
                    table_hbm.at[idx_v.at[pl.ds(j * _SUB, _SUB)]],
                    rows_v.at[slot].at[pl.ds(j * _SUB, _SUB)],
                    sem,
                ).wait()

        fire(0, 0, sems[0])

        def pair(p, carry):
            for b in range(2):
                i = p * 2 + b
                if b == 0:
                    fire(i + 1, 1, sems[1])
                else:
                    @pl.when(i + 1 < nchunks)
                    def _():
                        fire(i + 1, 0, sems[0])
                drain(b, sems[b])
                pltpu.sync_copy(rows_v.at[b], out_hbm.at[pl.ds(base + i * _CH, _CH)])
            return carry

        lax.fori_loop(0, nchunks // 2, pair, 0)

    return gather_kernel


def kernel(word_indices, embeddings):
    batch, hist = word_indices.shape
    vocab, d = embeddings.shape
    idx = word_indices.reshape(-1).astype(jnp.int32)
    out = _build_gather(batch * hist, vocab, d)(idx, embeddings)
    return out.reshape(batch, hist, d)
